# baseline (speedup 1.0000x reference)
import jax
import jax.numpy as jnp
from jax.experimental import pallas as pl
from jax.experimental.pallas import tpu as pltpu

_BR = 200
_PAIR = 2 * _BR


def _gcn_kernel(x_ref, adja_ref, adjb_ref, w1_ref, b1_ref, w2_ref, b2_ref,
                x1_ref, out_ref, s1_scr, s2_scr, x1_scr):
    p = pl.program_id(0)
    i = pl.program_id(1)

    @pl.when(jnp.logical_and(p == 0, i == 0))
    def _():
        s1_scr[...] = jnp.dot(x_ref[...], w1_ref[...],
                              preferred_element_type=jnp.float32)

    @pl.when(p == 0)
    def _():
        ha = jnp.dot(adja_ref[...], s1_scr[...],
                     preferred_element_type=jnp.float32)
        hb = jnp.dot(adjb_ref[...], s1_scr[...],
                     preferred_element_type=jnp.float32)
        x1 = jnp.maximum(jnp.concatenate([ha, hb], axis=0) + b1_ref[...],
                         0.0)
        x1_scr[pl.ds(i * _PAIR, _PAIR), :] = x1
        x1_ref[...] = x1
        s2_scr[pl.ds(i * _PAIR, _PAIR), :] = jnp.dot(
            x1, w2_ref[...], preferred_element_type=jnp.float32)

    @pl.when(p == 1)
    def _():
        ha = jnp.dot(adja_ref[...], s2_scr[...],
                     preferred_element_type=jnp.float32)
        hb = jnp.dot(adjb_ref[...], s2_scr[...],
                     preferred_element_type=jnp.float32)
        h2 = jnp.concatenate([ha, hb], axis=0) + b2_ref[...]
        m = jnp.max(h2, axis=1, keepdims=True)
        lse = jnp.log(jnp.sum(jnp.exp(h2 - m), axis=1, keepdims=True)) + m
        out_ref[...] = h2 - lse
        x1_ref[...] = x1_scr[pl.ds(i * _PAIR, _PAIR), :]


def kernel(x, adj, W1, b1, W2, b2):
    n, d_in = x.shape
    d_hid = W1.shape[1]
    d_out = W2.shape[1]
    nb = n // _PAIR

    x1, out = pl.pallas_call(
        _gcn_kernel,
        grid=(2, nb),
        in_specs=[
            pl.BlockSpec((n, d_in), lambda p, i: (0, 0)),
            pl.BlockSpec((_BR, n), lambda p, i: (2 * i, 0)),
            pl.BlockSpec((_BR, n), lambda p, i: (2 * i + 1, 0)),
            pl.BlockSpec((d_in, d_hid), lambda p, i: (0, 0)),
            pl.BlockSpec((1, d_hid), lambda p, i: (0, 0)),
            pl.BlockSpec((d_hid, d_out), lambda p, i: (0, 0)),
            pl.BlockSpec((1, d_out), lambda p, i: (0, 0)),
        ],
        out_specs=[
            pl.BlockSpec((_PAIR, d_hid), lambda p, i: (i, 0)),
            pl.BlockSpec((_PAIR, d_out), lambda p, i: (i, 0)),
        ],
        out_shape=[
            jax.ShapeDtypeStruct((n, d_hid), jnp.float32),
            jax.ShapeDtypeStruct((n, d_out), jnp.float32),
        ],
        scratch_shapes=[
            pltpu.VMEM((n, d_hid), jnp.float32),
            pltpu.VMEM((n, d_out), jnp.float32),
            pltpu.VMEM((n, d_hid), jnp.float32),
        ],
    )(x, adj, adj, W1, b1.reshape(1, d_hid), W2, b2.reshape(1, d_out))

    return (out, x1)
